# baseline (speedup 1.0000x reference)
import jax
import jax.numpy as jnp
from jax import lax
from jax.experimental import pallas as pl
from jax.experimental.pallas import tpu as pltpu
from jax.experimental.pallas import tpu_sc as plsc

B, N, Q = 2048, 8192, 2048
L = 16
NC, NS = 2, 16
NW = NC * NS
ROWS_PER_W = B // NW
RPD = 2
GROUPS = ROWS_PER_W // RPD
NBUF = 4


def _tec_body(x_hbm, y_hbm, out_hbm, xv, idxv, wv, rowbuf, outbuf,
              sem_in0, sem_in1, sem_in2, sem_in3,
              sem_out0, sem_out1, sem_out2, sem_out3):
    sems_in = (sem_in0, sem_in1, sem_in2, sem_in3)
    sems_out = (sem_out0, sem_out1, sem_out2, sem_out3)
    wid = lax.axis_index("s") * NC + lax.axis_index("c")
    base_row = wid * ROWS_PER_W

    def in_copy(b, row):
        return pltpu.make_async_copy(
            y_hbm.at[pl.ds(row, RPD)], rowbuf.at[pl.ds(b * RPD, RPD)],
            sems_in[b])

    def out_copy(b, row):
        return pltpu.make_async_copy(
            outbuf.at[pl.ds(b * RPD, RPD)], out_hbm.at[pl.ds(row, RPD)],
            sems_out[b])

    for b in range(NBUF):
        in_copy(b, base_row + b * RPD).start()

    pltpu.sync_copy(x_hbm, xv)
    scale = jnp.float32(N - 1)

    @plsc.parallel_loop(0, Q // L, unroll=4)
    def _idx_loop(i):
        x = xv[pl.ds(i * L, L)]
        t = x * scale
        idx = t.astype(jnp.int32)
        idx = jnp.minimum(jnp.maximum(idx, 0), N - 2)
        w = t - idx.astype(jnp.float32)
        idxv[pl.ds(i * L, L)] = idx
        wv[pl.ds(i * L, L)] = w

    def group_body(g, carry):
        for b in range(NBUF):
            gi = g * NBUF + b
            row = base_row + gi * RPD
            in_copy(b, row).wait()

            @pl.when(g > 0)
            def _():
                out_copy(b, row - NBUF * RPD).wait()

            @plsc.parallel_loop(0, Q // L, unroll=4)
            def _q_loop(c):
                iv = idxv[pl.ds(c * L, L)]
                w = wv[pl.ds(c * L, L)]
                for j in range(RPD):
                    rj = jnp.full((L,), b * RPD + j, jnp.int32)
                    y1 = plsc.load_gather(rowbuf, [rj, iv])
                    y2 = plsc.load_gather(rowbuf, [rj, iv + 1])
                    outbuf[b * RPD + j, pl.ds(c * L, L)] = y1 + w * (y2 - y1)

            out_copy(b, row).start()

            @pl.when(gi + NBUF < GROUPS)
            def _():
                in_copy(b, row + NBUF * RPD).start()
        return carry

    lax.fori_loop(0, GROUPS // NBUF, group_body, 0)

    for b in range(NBUF):
        out_copy(b, base_row + (GROUPS - NBUF + b) * RPD).wait()


def kernel(x_new_, y_points):
    mesh = plsc.VectorSubcoreMesh(core_axis_name="c", subcore_axis_name="s")
    k = pl.kernel(
        _tec_body,
        out_type=jax.ShapeDtypeStruct((B, Q), jnp.float32),
        mesh=mesh,
        compiler_params=pltpu.CompilerParams(needs_layout_passes=False),
        scratch_types=[
            pltpu.VMEM((Q,), jnp.float32),
            pltpu.VMEM((Q,), jnp.int32),
            pltpu.VMEM((Q,), jnp.float32),
            pltpu.VMEM((NBUF * RPD, N), jnp.float32),
            pltpu.VMEM((NBUF * RPD, Q), jnp.float32),
            pltpu.SemaphoreType.DMA,
            pltpu.SemaphoreType.DMA,
            pltpu.SemaphoreType.DMA,
            pltpu.SemaphoreType.DMA,
            pltpu.SemaphoreType.DMA,
            pltpu.SemaphoreType.DMA,
            pltpu.SemaphoreType.DMA,
            pltpu.SemaphoreType.DMA,
        ],
    )
    return k(x_new_, y_points)


# speedup vs baseline: 1.0035x; 1.0035x over previous
import jax
import jax.numpy as jnp
from jax import lax
from jax.experimental import pallas as pl
from jax.experimental.pallas import tpu as pltpu
from jax.experimental.pallas import tpu_sc as plsc

B, N, Q = 2048, 8192, 2048
L = 16
NC, NS = 2, 16
NW = NC * NS

B_TC = 512
B_SC = B - B_TC
ROWS_PER_W = B_SC // NW
NBUF = 4

NK = 512
K_STEPS = N // NK


def _sc_body(x_hbm, y_hbm, out_hbm, xv, idxv, wv, rowbuf, outbuf, *sems):
    sems_in = sems[:NBUF]
    sems_out = sems[NBUF:]
    wid = lax.axis_index("s") * NC + lax.axis_index("c")
    base_row = B_TC + wid * ROWS_PER_W

    def in_copy(b, row):
        return pltpu.make_async_copy(
            y_hbm.at[row], rowbuf.at[pl.ds(b * N, N)], sems_in[b])

    def out_copy(b, row):
        return pltpu.make_async_copy(
            outbuf.at[pl.ds(b * Q, Q)], out_hbm.at[row], sems_out[b])

    for b in range(NBUF):
        in_copy(b, base_row + b).start()

    pltpu.sync_copy(x_hbm, xv)
    scale = jnp.float32(N - 1)

    @plsc.parallel_loop(0, Q // L, unroll=4)
    def _idx_loop(i):
        x = xv[pl.ds(i * L, L)]
        t = x * scale
        idx = t.astype(jnp.int32)
        idx = jnp.minimum(jnp.maximum(idx, 0), N - 2)
        w = t - idx.astype(jnp.float32)
        idxv[pl.ds(i * L, L)] = idx
        wv[pl.ds(i * L, L)] = w

    def group_body(g, carry):
        for b in range(NBUF):
            r = g * NBUF + b
            row = base_row + r
            in_copy(b, row).wait()

            @pl.when(g > 0)
            def _():
                out_copy(b, row - NBUF).wait()

            boff = b * N

            @plsc.parallel_loop(0, Q // L, unroll=4)
            def _q_loop(c):
                iv = idxv[pl.ds(c * L, L)] + boff
                w = wv[pl.ds(c * L, L)]
                y1 = plsc.load_gather(rowbuf, [iv])
                y2 = plsc.load_gather(rowbuf, [iv + 1])
                outbuf[pl.ds(b * Q + c * L, L)] = y1 + w * (y2 - y1)

            out_copy(b, row).start()

            @pl.when(r + NBUF < ROWS_PER_W)
            def _():
                in_copy(b, row + NBUF).start()
        return carry

    lax.fori_loop(0, ROWS_PER_W // NBUF, group_body, 0)

    for b in range(NBUF):
        out_copy(b, base_row + ROWS_PER_W - NBUF + b).wait()


def _sc_interp(x_new_, y_points):
    mesh = plsc.VectorSubcoreMesh(core_axis_name="c", subcore_axis_name="s")
    k = pl.kernel(
        _sc_body,
        out_type=jax.ShapeDtypeStruct((B, Q), jnp.float32),
        mesh=mesh,
        compiler_params=pltpu.CompilerParams(needs_layout_passes=False),
        scratch_types=[
            pltpu.VMEM((Q,), jnp.float32),
            pltpu.VMEM((Q,), jnp.int32),
            pltpu.VMEM((Q,), jnp.float32),
            pltpu.VMEM((NBUF * N,), jnp.float32),
            pltpu.VMEM((NBUF * Q,), jnp.float32),
        ] + [pltpu.SemaphoreType.DMA] * (2 * NBUF),
    )
    return k(x_new_, y_points)


def _tc_body(x_ref, y_ref, out_ref):
    k = pl.program_id(0)
    t = x_ref[...] * jnp.float32(N - 1)
    n = lax.broadcasted_iota(jnp.int32, (NK, Q), 0) + k * NK
    s = jnp.maximum(1.0 - jnp.abs(t - n.astype(jnp.float32)), 0.0)
    s16 = s.astype(jnp.bfloat16)
    y16 = y_ref[...].astype(jnp.bfloat16)
    part = lax.dot_general(y16, s16, (((1,), (0,)), ((), ())),
                           preferred_element_type=jnp.float32)

    @pl.when(k == 0)
    def _():
        out_ref[...] = part

    @pl.when(k > 0)
    def _():
        out_ref[...] += part


def _tc_interp(x_new_, y_points):
    return pl.pallas_call(
        _tc_body,
        grid=(K_STEPS,),
        in_specs=[
            pl.BlockSpec((1, Q), lambda k: (0, 0)),
            pl.BlockSpec((B_TC, NK), lambda k: (0, k)),
        ],
        out_specs=pl.BlockSpec((B_TC, Q), lambda k: (0, 0)),
        out_shape=jax.ShapeDtypeStruct((B_TC, Q), jnp.float32),
        compiler_params=pltpu.CompilerParams(
            dimension_semantics=("arbitrary",)),
    )(x_new_.reshape(1, Q), y_points)


def kernel(x_new_, y_points):
    sc_out = _sc_interp(x_new_, y_points)
    tc_out = _tc_interp(x_new_, y_points)
    return lax.dynamic_update_slice(sc_out, tc_out, (0, 0))


# speedup vs baseline: 1.0038x; 1.0003x over previous
import jax
import jax.numpy as jnp
from jax import lax
from jax.experimental import pallas as pl
from jax.experimental.pallas import tpu as pltpu
from jax.experimental.pallas import tpu_sc as plsc

B, N, Q = 2048, 8192, 2048
L = 16
NC, NS = 2, 16
NW = NC * NS

B_TC = 512
B_SC = B - B_TC
ROWS_PER_W = B_SC // NW
NBUF = 4

NK = 512
K_STEPS = N // NK


def _sc_body(x_hbm, y_hbm, out_hbm, xv, idxv, wv, rowbuf, outbuf, *sems):
    sems_in = sems[:NBUF]
    sems_out = sems[NBUF:]
    wid = lax.axis_index("s") * NC + lax.axis_index("c")
    base_row = B_TC + wid * ROWS_PER_W

    def in_copy(b, row):
        return pltpu.make_async_copy(
            y_hbm.at[row], rowbuf.at[pl.ds(b * N, N)], sems_in[b])

    def out_copy(b, row):
        return pltpu.make_async_copy(
            outbuf.at[pl.ds(b * Q, Q)], out_hbm.at[row], sems_out[b])

    for b in range(NBUF):
        in_copy(b, base_row + b).start()

    pltpu.sync_copy(x_hbm, xv)
    scale = jnp.float32(N - 1)

    @plsc.parallel_loop(0, Q // L, unroll=4)
    def _idx_loop(i):
        x = xv[pl.ds(i * L, L)]
        t = x * scale
        idx = t.astype(jnp.int32)
        idx = jnp.minimum(jnp.maximum(idx, 0), N - 2)
        w = t - idx.astype(jnp.float32)
        idxv[pl.ds(i * L, L)] = idx
        wv[pl.ds(i * L, L)] = w

    def group_body(g, carry):
        for b in range(NBUF):
            r = g * NBUF + b
            row = base_row + r
            in_copy(b, row).wait()

            @pl.when(g > 0)
            def _():
                out_copy(b, row - NBUF).wait()

            boff = b * N

            @plsc.parallel_loop(0, Q // L, unroll=4)
            def _q_loop(c):
                iv = idxv[pl.ds(c * L, L)] + boff
                w = wv[pl.ds(c * L, L)]
                y1 = plsc.load_gather(rowbuf, [iv])
                y2 = plsc.load_gather(rowbuf, [iv + 1])
                outbuf[pl.ds(b * Q + c * L, L)] = y1 + w * (y2 - y1)

            out_copy(b, row).start()

            @pl.when(r + NBUF < ROWS_PER_W)
            def _():
                in_copy(b, row + NBUF).start()
        return carry

    lax.fori_loop(0, ROWS_PER_W // NBUF, group_body, 0)

    for b in range(NBUF):
        out_copy(b, base_row + ROWS_PER_W - NBUF + b).wait()


def _sc_interp(x_new_, y_points):
    mesh = plsc.VectorSubcoreMesh(core_axis_name="c", subcore_axis_name="s")
    k = pl.kernel(
        _sc_body,
        out_type=jax.ShapeDtypeStruct((B, Q), jnp.float32),
        mesh=mesh,
        compiler_params=pltpu.CompilerParams(needs_layout_passes=False),
        scratch_types=[
            pltpu.VMEM((Q,), jnp.float32),
            pltpu.VMEM((Q,), jnp.int32),
            pltpu.VMEM((Q,), jnp.float32),
            pltpu.VMEM((NBUF * N,), jnp.float32),
            pltpu.VMEM((NBUF * Q,), jnp.float32),
        ] + [pltpu.SemaphoreType.DMA] * (2 * NBUF),
    )
    return k(x_new_, y_points)


def _tc_body(x_ref, y_ref, out_ref):
    k = pl.program_id(0)
    t = x_ref[...] * jnp.float32(N - 1)
    n = lax.broadcasted_iota(jnp.int32, (NK, Q), 0) + k * NK
    s = jnp.maximum(1.0 - jnp.abs(t - n.astype(jnp.float32)), 0.0)
    s16 = s.astype(jnp.bfloat16)
    y16 = y_ref[...].astype(jnp.bfloat16)
    part = lax.dot_general(y16, s16, (((1,), (0,)), ((), ())),
                           preferred_element_type=jnp.float32)

    @pl.when(k == 0)
    def _():
        out_ref[...] = part

    @pl.when(k > 0)
    def _():
        out_ref[...] += part


def _tc_interp(x_new_, y_points):
    return pl.pallas_call(
        _tc_body,
        grid=(K_STEPS,),
        in_specs=[
            pl.BlockSpec((1, Q), lambda k: (0, 0)),
            pl.BlockSpec((B_TC, NK), lambda k: (0, k)),
        ],
        out_specs=pl.BlockSpec((B_TC, Q), lambda k: (0, 0)),
        out_shape=jax.ShapeDtypeStruct((B_TC, Q), jnp.float32),
        compiler_params=pltpu.CompilerParams(
            dimension_semantics=("arbitrary",)),
    )(x_new_.reshape(1, Q), y_points)


def kernel(x_new_, y_points):
    tc_out = _tc_interp(x_new_, y_points)
    sc_out = _sc_interp(x_new_, y_points)
    return lax.dynamic_update_slice(sc_out, tc_out, (0, 0))


# speedup vs baseline: 1.0179x; 1.0140x over previous
import jax
import jax.numpy as jnp
from jax import lax
from jax.experimental import pallas as pl
from jax.experimental.pallas import tpu as pltpu
from jax.experimental.pallas import tpu_sc as plsc

B, N, Q = 2048, 8192, 2048
L = 16
NC, NS = 2, 16
NW = NC * NS
ROWS_PER_W = B // NW
NBUF = 4
ROUNDS = ROWS_PER_W // NBUF


def _tec_body(x_hbm, y_hbm, out_hbm, xv, idxv, wv, rowbuf, outbuf, spbuf,
              *sems):
    sems_in = sems[:NBUF]
    sems_mid = sems[NBUF:2 * NBUF]
    sems_out = sems[2 * NBUF:]
    cid = lax.axis_index("c")
    sid = lax.axis_index("s")
    wid = sid * NC + cid
    base_row = wid * ROWS_PER_W

    def in_copy(b, row):
        return pltpu.make_async_copy(
            y_hbm.at[row], rowbuf.at[pl.ds(b * N, N)], sems_in[b])

    def mid_copy(b, par):
        return pltpu.make_async_copy(
            outbuf.at[pl.ds(b * Q, Q)], spbuf.at[sid, b, par], sems_mid[b])

    def out_copy(b, par, row):
        return pltpu.make_async_copy(
            spbuf.at[sid, b, par], out_hbm.at[row], sems_out[b])

    for b in range(NBUF):
        in_copy(b, base_row + b).start()

    pltpu.sync_copy(x_hbm, xv)
    scale = jnp.float32(N - 1)

    @plsc.parallel_loop(0, Q // L, unroll=4)
    def _idx_loop(i):
        x = xv[pl.ds(i * L, L)]
        t = x * scale
        idx = t.astype(jnp.int32)
        idx = jnp.minimum(jnp.maximum(idx, 0), N - 2)
        w = t - idx.astype(jnp.float32)
        idxv[pl.ds(i * L, L)] = idx
        wv[pl.ds(i * L, L)] = w

    def round_body(g, carry):
        par = lax.rem(g, 2)
        for b in range(NBUF):
            row = base_row + g * NBUF + b
            in_copy(b, row).wait()

            @pl.when(g > 0)
            def _():
                mid_copy(b, 1 - par).wait()
                out_copy(b, 1 - par, row - NBUF).start()

            @pl.when(g > 1)
            def _():
                out_copy(b, par, row - 2 * NBUF).wait()

            boff = b * N

            @plsc.parallel_loop(0, Q // L, unroll=4)
            def _q_loop(c):
                iv = idxv[pl.ds(c * L, L)] + boff
                w = wv[pl.ds(c * L, L)]
                y1 = plsc.load_gather(rowbuf, [iv])
                y2 = plsc.load_gather(rowbuf, [iv + 1])
                outbuf[pl.ds(b * Q + c * L, L)] = y1 + w * (y2 - y1)

            mid_copy(b, par).start()

            @pl.when(row + NBUF < base_row + ROWS_PER_W)
            def _():
                in_copy(b, row + NBUF).start()
        return carry

    lax.fori_loop(0, ROUNDS, round_body, 0)

    last_par = (ROUNDS - 1) % 2
    for b in range(NBUF):
        last_row = base_row + (ROUNDS - 1) * NBUF + b
        mid_copy(b, last_par).wait()
        out_copy(b, last_par, last_row).start()
        if ROUNDS > 1:
            out_copy(b, 1 - last_par, last_row - NBUF).wait()
        out_copy(b, last_par, last_row).wait()


def kernel(x_new_, y_points):
    mesh = plsc.VectorSubcoreMesh(core_axis_name="c", subcore_axis_name="s")
    k = pl.kernel(
        _tec_body,
        out_type=jax.ShapeDtypeStruct((B, Q), jnp.float32),
        mesh=mesh,
        compiler_params=pltpu.CompilerParams(needs_layout_passes=False),
        scratch_types=[
            pltpu.VMEM((Q,), jnp.float32),
            pltpu.VMEM((Q,), jnp.int32),
            pltpu.VMEM((Q,), jnp.float32),
            pltpu.VMEM((NBUF * N,), jnp.float32),
            pltpu.VMEM((NBUF * Q,), jnp.float32),
            pltpu.VMEM_SHARED((NS, NBUF, 2, Q), jnp.float32),
        ] + [pltpu.SemaphoreType.DMA] * (3 * NBUF),
    )
    return k(x_new_, y_points)


# speedup vs baseline: 1.0379x; 1.0197x over previous
import jax
import jax.numpy as jnp
from jax import lax
from jax.experimental import pallas as pl
from jax.experimental.pallas import tpu as pltpu
from jax.experimental.pallas import tpu_sc as plsc

B, N, Q = 2048, 8192, 2048
L = 16
NC, NS = 2, 16
NW = NC * NS
ROWS_PER_W = B // NW
NBUF = 4


def _tec_body(x_hbm, y_hbm, out_hbm, xv, idxv, wv, rowbuf, outbuf, *sems):
    sems_in = sems[:NBUF]
    sems_out = sems[NBUF:]
    wid = lax.axis_index("s") * NC + lax.axis_index("c")
    base_row = wid * ROWS_PER_W

    def in_copy(b, row):
        return pltpu.make_async_copy(
            y_hbm.at[row], rowbuf.at[pl.ds(b * N, N)], sems_in[b])

    def out_copy(b, row):
        return pltpu.make_async_copy(
            outbuf.at[pl.ds(b * Q, Q)], out_hbm.at[row], sems_out[b])

    for b in range(NBUF):
        in_copy(b, base_row + b).start()

    pltpu.sync_copy(x_hbm, xv)
    scale = jnp.float32(N - 1)

    @plsc.parallel_loop(0, Q // L, unroll=4)
    def _idx_loop(i):
        x = xv[pl.ds(i * L, L)]
        t = x * scale
        idx = t.astype(jnp.int32)
        idx = jnp.minimum(jnp.maximum(idx, 0), N - 2)
        w = t - idx.astype(jnp.float32)
        idxv[pl.ds(i * L, L)] = idx
        wv[pl.ds(i * L, L)] = w

    def group_body(g, carry):
        for b in range(NBUF):
            r = g * NBUF + b
            row = base_row + r
            in_copy(b, row).wait()

            @pl.when(g > 0)
            def _():
                out_copy(b, row - NBUF).wait()

            boff = b * N

            @plsc.parallel_loop(0, Q // L, unroll=8)
            def _q_loop(c):
                iv = idxv[pl.ds(c * L, L)] + boff
                w = wv[pl.ds(c * L, L)]
                y1 = plsc.load_gather(rowbuf, [iv])
                y2 = plsc.load_gather(rowbuf, [iv + 1])
                outbuf[pl.ds(b * Q + c * L, L)] = y1 + w * (y2 - y1)

            out_copy(b, row).start()

            @pl.when(r + NBUF < ROWS_PER_W)
            def _():
                in_copy(b, row + NBUF).start()
        return carry

    lax.fori_loop(0, ROWS_PER_W // NBUF, group_body, 0)

    for b in range(NBUF):
        out_copy(b, base_row + ROWS_PER_W - NBUF + b).wait()


def kernel(x_new_, y_points):
    mesh = plsc.VectorSubcoreMesh(core_axis_name="c", subcore_axis_name="s")
    k = pl.kernel(
        _tec_body,
        out_type=jax.ShapeDtypeStruct((B, Q), jnp.float32),
        mesh=mesh,
        compiler_params=pltpu.CompilerParams(needs_layout_passes=False),
        scratch_types=[
            pltpu.VMEM((Q,), jnp.float32),
            pltpu.VMEM((Q,), jnp.int32),
            pltpu.VMEM((Q,), jnp.float32),
            pltpu.VMEM((NBUF * N,), jnp.float32),
            pltpu.VMEM((NBUF * Q,), jnp.float32),
        ] + [pltpu.SemaphoreType.DMA] * (2 * NBUF),
    )
    return k(x_new_, y_points)
